# Initial kernel scaffold; baseline (speedup 1.0000x reference)
import jax
import jax.numpy as jnp
from jax.experimental import pallas as pl


def kernel(x, edge_index, batch, W1_rel, b1, W1_root, p1_w, W2_rel, b2, W2_root, p2_w, W3_rel, b3, W3_root, p3_w, L1_W, L1_b, L2_W, L2_b, L3_W, L3_b):
    raise NotImplementedError("write your pallas kernel here")


# speedup vs baseline: 20.1596x; 20.1596x over previous
import functools
import jax
import jax.numpy as jnp
from jax import lax
from jax.experimental import pallas as pl
from jax.experimental.pallas import tpu as pltpu
from jax.experimental.pallas import tpu_sc as plsc

NNODE = 10000
NPAD = 10240
NROW = NPAD // 128
EDGES = 320000
D = 128
NC, NS = 2, 16
EPW = EDGES // (NC * NS)
CHUNK = 80
NCHUNK = EPW // CHUNK
ZROWS = NPAD // NS
ZBUF = 64
INT_MIN = -2147483648


def _sc_agg_body(x_hbm, src_hbm, dst_hbm, out_hbm,
                 sidx, didx, rows, zbuf, acc, sem):
    c = lax.axis_index("c")
    s = lax.axis_index("s")

    zv = jnp.zeros((16,), jnp.float32)

    def zb(i, carry):
        r = i // (D // 16)
        l = (i % (D // 16)) * 16
        zbuf[r, pl.ds(l, 16)] = zv
        return carry

    lax.fori_loop(0, ZBUF * (D // 16), zb, 0)

    def zs(i, carry):
        pltpu.sync_copy(zbuf, acc.at[pl.ds(s * ZROWS + i * ZBUF, ZBUF)])
        return carry

    lax.fori_loop(0, ZROWS // ZBUF, zs, 0)
    plsc.subcore_barrier()

    wstart = (c * NS + s) * EPW

    def step(t, carry):
        base = wstart + t * CHUNK
        pltpu.sync_copy(src_hbm.at[pl.ds(base, CHUNK)], sidx)
        pltpu.sync_copy(dst_hbm.at[pl.ds(base, CHUNK)], didx)
        pltpu.async_copy(x_hbm.at[sidx], rows, sem).wait()
        pltpu.sync_copy(rows, acc.at[didx], add=True)
        return carry

    lax.fori_loop(0, NCHUNK, step, 0)
    plsc.subcore_barrier()

    pltpu.sync_copy(acc.at[pl.ds(s * ZROWS, ZROWS)],
                    out_hbm.at[c, pl.ds(s * ZROWS, ZROWS)])


def _sc_agg(x_pad, src, dst):
    mesh = plsc.VectorSubcoreMesh(core_axis_name="c", subcore_axis_name="s")
    f = pl.kernel(
        _sc_agg_body,
        out_type=jax.ShapeDtypeStruct((NC, NPAD, D), jnp.float32),
        mesh=mesh,
        scratch_types=[
            pltpu.VMEM((CHUNK,), jnp.int32),
            pltpu.VMEM((CHUNK,), jnp.int32),
            pltpu.VMEM((CHUNK, D), jnp.float32),
            pltpu.VMEM((ZBUF, D), jnp.float32),
            pltpu.VMEM_SHARED((NPAD, D), jnp.float32),
            pltpu.SemaphoreType.DMA,
        ],
    )
    return f(x_pad, src, dst)


def _sortable(s):
    b = jax.lax.bitcast_convert_type(s, jnp.int32)
    return jnp.where(b >= 0, b, b ^ jnp.int32(0x7FFFFFFF))


def _count(m):
    return jnp.sum(m.astype(jnp.int32))


def _radix_kth(key, live, need):
    imin = jnp.int32(INT_MIN)

    def body(i, prefix):
        cand = prefix | (jnp.int32(1) << (jnp.int32(31) - i))
        scand = cand ^ imin
        cnt = _count(live & (key >= scand))
        return jnp.where(cnt >= need, cand, prefix)

    prefix = lax.fori_loop(0, 32, body, jnp.int32(0))
    return prefix ^ imin


def _tc_conv_body(p0, p1, X, Wrel, brel, Wroot, pw, H_o, score_o):
    agg = p0[...] + p1[...]
    H = agg @ Wrel[...] + brel[...] + X[...] @ Wroot[...]
    H = jnp.maximum(H, 0.0)
    H_o[...] = H
    pwv = pw[...]
    rnorm = 1.0 / (jnp.sqrt(jnp.sum(pwv * pwv)) + 1e-16)
    score_o[...] = jnp.tanh((H @ pwv.reshape(D, 1)) * rnorm)


def _tc_conv(p0, p1, X, Wrel, brel, Wroot, pw):
    return pl.pallas_call(
        _tc_conv_body,
        out_shape=[jax.ShapeDtypeStruct((NPAD, D), jnp.float32),
                   jax.ShapeDtypeStruct((NPAD, 1), jnp.float32)],
    )(p0, p1, X, Wrel, brel, Wroot, pw)


def _tc_select_body(nhist, k, *refs):
    (s80, mask80, idx80) = refs[:3]
    hist = refs[3:3 + nhist]
    (s80m_o, sel80_o) = refs[3 + nhist:]

    live = mask80[...] != 0
    s80m = jnp.where(live, s80[...], -jnp.inf)
    s80m_o[...] = s80m

    need = jnp.int32(k)
    sel = jnp.zeros((NROW, D), jnp.bool_)
    for lvl in range(nhist + 1):
        sl = s80m if lvl == 0 else hist[lvl - 1][...]
        key = _sortable(jnp.where(live, sl, -jnp.inf))
        K = _radix_kth(key, live, need)
        gt = live & (key > K)
        sel = sel | gt
        need = need - _count(gt)
        live = live & (key == K)

    idx = idx80[...]

    def ibody(i, lohi):
        lo, hi = lohi
        mid = (lo + hi) // 2
        cnt = _count(live & (idx < mid))
        return (jnp.where(cnt >= need, lo, mid + 1),
                jnp.where(cnt >= need, mid, hi))

    lo, hi = lax.fori_loop(0, 15, ibody, (jnp.int32(0), jnp.int32(NPAD)))
    sel = sel | (live & (idx < lo))
    sel80_o[...] = sel.astype(jnp.float32)


def _tc_select(nhist, k, s80, mask80, idx80, hist):
    body = functools.partial(_tc_select_body, nhist, k)
    return pl.pallas_call(
        body,
        out_shape=[jax.ShapeDtypeStruct((NROW, D), jnp.float32),
                   jax.ShapeDtypeStruct((NROW, D), jnp.float32)],
    )(s80, mask80, idx80, *hist)


def _tc_apply_body(H, score, selc, Xn_o, xmax_o, xsum_o):
    keep = selc[...] != 0.0
    Xn = jnp.where(keep, H[...] * score[...], 0.0)
    Xn_o[...] = Xn
    xmax_o[...] = jnp.max(jnp.where(keep, Xn, -jnp.inf), axis=0, keepdims=True)
    xsum_o[...] = jnp.sum(Xn, axis=0, keepdims=True)


def _tc_apply(H, score, selc):
    return pl.pallas_call(
        _tc_apply_body,
        out_shape=[jax.ShapeDtypeStruct((NPAD, D), jnp.float32),
                   jax.ShapeDtypeStruct((1, D), jnp.float32),
                   jax.ShapeDtypeStruct((1, D), jnp.float32)],
    )(H, score, selc)


def _tc_head_body(hA, hB, L1a, L1b_, L1bias, L2W, L2b, L3W, L3b, out):
    h = hA[...] @ L1a[...] + hB[...] @ L1b_[...] + L1bias[...]
    h = jnp.maximum(h, 0.0)
    h = jnp.maximum(h @ L2W[...] + L2b[...], 0.0)
    out[...] = h @ L3W[...] + L3b[...]


def _tc_head(hA, hB, L1a, L1b_, L1bias, L2W, L2b, L3W, L3b):
    return pl.pallas_call(
        _tc_head_body,
        out_shape=jax.ShapeDtypeStruct((1, 1), jnp.float32),
    )(hA, hB, L1a, L1b_, L1bias, L2W, L2b, L3W, L3b)


def kernel(x, edge_index, batch, W1_rel, b1, W1_root, p1_w, W2_rel, b2,
           W2_root, p2_w, W3_rel, b3, W3_root, p3_w, L1_W, L1_b, L2_W, L2_b,
           L3_W, L3_b):
    src = edge_index[0]
    dst = edge_index[1]

    X = jnp.pad(x, ((0, NPAD - NNODE), (0, 0)))
    idx_flat = jnp.arange(NPAD, dtype=jnp.int32)
    idx80 = idx_flat.reshape(NROW, D)
    mask80 = (idx_flat < NNODE).astype(jnp.float32).reshape(NROW, D)

    ks = (8000, 6400, 5120)
    rounds = ((W1_rel, b1.reshape(1, D), W1_root, p1_w.reshape(1, D)),
              (W2_rel, b2.reshape(1, D), W2_root, p2_w.reshape(1, D)),
              (W3_rel, b3.reshape(1, D), W3_root, p3_w.reshape(1, D)))

    hist = []
    pooled = []
    for r in range(3):
        Wrel, brel, Wroot, pw = rounds[r]
        parts = _sc_agg(X, src, dst)
        H, score = _tc_conv(parts[0], parts[1], X, Wrel, brel, Wroot, pw)
        s80 = score.reshape(NROW, D)
        s80m, sel80 = _tc_select(r, ks[r], s80, mask80, idx80, hist)
        selc = sel80.reshape(NPAD, 1)
        X, xmax, xsum = _tc_apply(H, score, selc)
        hist = [s80m] + hist
        mask80 = sel80
        pooled.append((xmax, xsum / ks[r]))

    hA = pooled[0][0] + pooled[1][0] + pooled[2][0]
    hB = pooled[0][1] + pooled[1][1] + pooled[2][1]
    out = _tc_head(hA, hB, L1_W[:D], L1_W[D:], L1_b.reshape(1, D),
                   L2_W, L2_b.reshape(1, D // 2), L3_W, L3_b.reshape(1, 1))
    return out.reshape(-1)


# speedup vs baseline: 27.2348x; 1.3510x over previous
import functools
import jax
import jax.numpy as jnp
from jax import lax
from jax.experimental import pallas as pl
from jax.experimental.pallas import tpu as pltpu
from jax.experimental.pallas import tpu_sc as plsc

NNODE = 10000
NPAD = 10240
NROW = NPAD // 128
EDGES = 320000
D = 128
NC, NS = 2, 16
EPW = EDGES // (NC * NS)
CHUNK = 80
NCHUNK = EPW // CHUNK
NSEC = 5
SECCHUNK = NCHUNK // NSEC
ZROWS = NPAD // NS
ZBUF = 32
INT_MIN = -2147483648


def _sc_agg_body(x_hbm, src_hbm, dst_hbm, out_hbm,
                 sidx, didx, rows0, rows1, zbuf, acc, isem, sem0, sem1):
    c = lax.axis_index("c")
    s = lax.axis_index("s")
    w = c * NS + s

    zv = jnp.zeros((16,), jnp.float32)

    def zb(i, carry):
        zbuf[i // (D // 16), pl.ds((i % (D // 16)) * 16, 16)] = zv
        return carry

    lax.fori_loop(0, ZBUF * (D // 16), zb, 0)

    def zs(i, carry):
        pltpu.sync_copy(zbuf, acc.at[pl.ds(s * ZROWS + i * ZBUF, ZBUF)])
        return carry

    lax.fori_loop(0, ZROWS // ZBUF, zs, 0)
    plsc.subcore_barrier()

    def gather(t, buf, sem):
        pltpu.async_copy(x_hbm.at[sidx.at[t]], buf, sem)

    def gwait(buf, sem):
        pltpu.make_async_copy(x_hbm.at[sidx.at[0]], buf, sem).wait()

    def scat(t, buf):
        pltpu.sync_copy(buf, acc.at[didx.at[t]], add=True)

    def section(sec, carry):
        pltpu.sync_copy(src_hbm.at[w, sec], sidx)
        pltpu.sync_copy(dst_hbm.at[w, sec], didx)
        def one(t, c2):
            gather(t, rows0, sem0)
            gwait(rows0, sem0)
            scat(t, rows0)
            return c2

        lax.fori_loop(0, SECCHUNK, one, 0)
        return carry

    lax.fori_loop(0, NSEC, section, 0)
    plsc.subcore_barrier()

    pltpu.sync_copy(acc.at[pl.ds(s * ZROWS, ZROWS)],
                    out_hbm.at[c, pl.ds(s * ZROWS, ZROWS)])


def _sc_agg(x_pad, src3, dst3):
    mesh = plsc.VectorSubcoreMesh(core_axis_name="c", subcore_axis_name="s")
    f = pl.kernel(
        _sc_agg_body,
        out_type=jax.ShapeDtypeStruct((NC, NPAD, D), jnp.float32),
        mesh=mesh,
        scratch_types=[
            pltpu.VMEM((SECCHUNK, CHUNK), jnp.int32),
            pltpu.VMEM((SECCHUNK, CHUNK), jnp.int32),
            pltpu.VMEM((CHUNK, D), jnp.float32),
            pltpu.VMEM((CHUNK, D), jnp.float32),
            pltpu.VMEM((ZBUF, D), jnp.float32),
            pltpu.VMEM_SHARED((NPAD, D), jnp.float32),
            pltpu.SemaphoreType.DMA,
            pltpu.SemaphoreType.DMA,
            pltpu.SemaphoreType.DMA,
        ],
    )
    return f(x_pad, src3, dst3)


def _sortable(s):
    b = jax.lax.bitcast_convert_type(s, jnp.int32)
    return jnp.where(b >= 0, b, b ^ jnp.int32(0x7FFFFFFF))


def _count(m):
    return jnp.sum(m.astype(jnp.int32))


def _radix_kth(key, live, need):
    imin = jnp.int32(INT_MIN)

    def body(i, prefix):
        cand = prefix | (jnp.int32(1) << (jnp.int32(31) - i))
        scand = cand ^ imin
        cnt = _count(live & (key >= scand))
        return jnp.where(cnt >= need, cand, prefix)

    prefix = lax.fori_loop(0, 32, body, jnp.int32(0))
    return prefix ^ imin


def _tc_conv_body(p0, p1, X, Wrel, brel, Wroot, pw, H_o, score_o):
    agg = p0[...] + p1[...]
    H = agg @ Wrel[...] + brel[...] + X[...] @ Wroot[...]
    H = jnp.maximum(H, 0.0)
    H_o[...] = H
    pwv = pw[...]
    rnorm = 1.0 / (jnp.sqrt(jnp.sum(pwv * pwv)) + 1e-16)
    score_o[...] = jnp.tanh((H @ pwv.reshape(D, 1)) * rnorm)


def _tc_conv(p0, p1, X, Wrel, brel, Wroot, pw):
    return pl.pallas_call(
        _tc_conv_body,
        out_shape=[jax.ShapeDtypeStruct((NPAD, D), jnp.float32),
                   jax.ShapeDtypeStruct((NPAD, 1), jnp.float32)],
    )(p0, p1, X, Wrel, brel, Wroot, pw)


def _tc_select_body(nhist, k, *refs):
    (s80, mask80, idx80) = refs[:3]
    hist = refs[3:3 + nhist]
    (s80m_o, sel80_o) = refs[3 + nhist:]

    live = mask80[...] != 0
    s80m = jnp.where(live, s80[...], -jnp.inf)
    s80m_o[...] = s80m

    need = jnp.int32(k)
    sel = jnp.zeros((NROW, D), jnp.bool_)
    for lvl in range(nhist + 1):
        sl = s80m if lvl == 0 else hist[lvl - 1][...]
        key = _sortable(jnp.where(live, sl, -jnp.inf))
        K = _radix_kth(key, live, need)
        gt = live & (key > K)
        sel = sel | gt
        need = need - _count(gt)
        live = live & (key == K)

    idx = idx80[...]

    def ibody(i, lohi):
        lo, hi = lohi
        mid = (lo + hi) // 2
        cnt = _count(live & (idx < mid))
        return (jnp.where(cnt >= need, lo, mid + 1),
                jnp.where(cnt >= need, mid, hi))

    lo, hi = lax.fori_loop(0, 15, ibody, (jnp.int32(0), jnp.int32(NPAD)))
    sel = sel | (live & (idx < lo))
    sel80_o[...] = sel.astype(jnp.float32)


def _tc_select(nhist, k, s80, mask80, idx80, hist):
    body = functools.partial(_tc_select_body, nhist, k)
    return pl.pallas_call(
        body,
        out_shape=[jax.ShapeDtypeStruct((NROW, D), jnp.float32),
                   jax.ShapeDtypeStruct((NROW, D), jnp.float32)],
    )(s80, mask80, idx80, *hist)


def _tc_apply_body(H, score, selc, Xn_o, xmax_o, xsum_o):
    keep = selc[...] != 0.0
    Xn = jnp.where(keep, H[...] * score[...], 0.0)
    Xn_o[...] = Xn
    xmax_o[...] = jnp.max(jnp.where(keep, Xn, -jnp.inf), axis=0, keepdims=True)
    xsum_o[...] = jnp.sum(Xn, axis=0, keepdims=True)


def _tc_apply(H, score, selc):
    return pl.pallas_call(
        _tc_apply_body,
        out_shape=[jax.ShapeDtypeStruct((NPAD, D), jnp.float32),
                   jax.ShapeDtypeStruct((1, D), jnp.float32),
                   jax.ShapeDtypeStruct((1, D), jnp.float32)],
    )(H, score, selc)


def _tc_head_body(hA, hB, L1a, L1b_, L1bias, L2W, L2b, L3W, L3b, out):
    h = hA[...] @ L1a[...] + hB[...] @ L1b_[...] + L1bias[...]
    h = jnp.maximum(h, 0.0)
    h = jnp.maximum(h @ L2W[...] + L2b[...], 0.0)
    out[...] = h @ L3W[...] + L3b[...]


def _tc_head(hA, hB, L1a, L1b_, L1bias, L2W, L2b, L3W, L3b):
    return pl.pallas_call(
        _tc_head_body,
        out_shape=jax.ShapeDtypeStruct((1, 1), jnp.float32),
    )(hA, hB, L1a, L1b_, L1bias, L2W, L2b, L3W, L3b)


def kernel(x, edge_index, batch, W1_rel, b1, W1_root, p1_w, W2_rel, b2,
           W2_root, p2_w, W3_rel, b3, W3_root, p3_w, L1_W, L1_b, L2_W, L2_b,
           L3_W, L3_b):
    src = edge_index[0].reshape(NC * NS, NSEC, SECCHUNK, CHUNK)
    dst = edge_index[1].reshape(NC * NS, NSEC, SECCHUNK, CHUNK)

    X = jnp.pad(x, ((0, NPAD - NNODE), (0, 0)))
    idx_flat = jnp.arange(NPAD, dtype=jnp.int32)
    idx80 = idx_flat.reshape(NROW, D)
    mask80 = (idx_flat < NNODE).astype(jnp.float32).reshape(NROW, D)

    ks = (8000, 6400, 5120)
    rounds = ((W1_rel, b1.reshape(1, D), W1_root, p1_w.reshape(1, D)),
              (W2_rel, b2.reshape(1, D), W2_root, p2_w.reshape(1, D)),
              (W3_rel, b3.reshape(1, D), W3_root, p3_w.reshape(1, D)))

    hist = []
    pooled = []
    for r in range(3):
        Wrel, brel, Wroot, pw = rounds[r]
        parts = _sc_agg(X, src, dst)
        H, score = _tc_conv(parts[0], parts[1], X, Wrel, brel, Wroot, pw)
        s80 = score.reshape(NROW, D)
        s80m, sel80 = _tc_select(r, ks[r], s80, mask80, idx80, hist)
        selc = sel80.reshape(NPAD, 1)
        X, xmax, xsum = _tc_apply(H, score, selc)
        hist = [s80m] + hist
        mask80 = sel80
        pooled.append((xmax, xsum / ks[r]))

    hA = pooled[0][0] + pooled[1][0] + pooled[2][0]
    hB = pooled[0][1] + pooled[1][1] + pooled[2][1]
    out = _tc_head(hA, hB, L1_W[:D], L1_W[D:], L1_b.reshape(1, D),
                   L2_W, L2_b.reshape(1, D // 2), L3_W, L3_b.reshape(1, 1))
    return out.reshape(-1)


# speedup vs baseline: 39.1658x; 1.4381x over previous
import functools
import jax
import jax.numpy as jnp
from jax import lax
from jax.experimental import pallas as pl
from jax.experimental.pallas import tpu as pltpu
from jax.experimental.pallas import tpu_sc as plsc

NNODE = 10000
NPAD = 10240
NROW = NPAD // 128
EDGES = 320000
D = 128
NC, NS = 2, 16
EPW = EDGES // (NC * NS)
CHUNK = 80
NCHUNK = EPW // CHUNK
NSEC = 5
SECCHUNK = NCHUNK // NSEC
ZROWS = NPAD // NS
ZBUF = 32
INT_MIN = -2147483648


def _sc_agg_body(x_hbm, src_hbm, dst_hbm, out_hbm,
                 sidx, didx, rows0, rows1, zbuf, acc, isem, sem0, sem1):
    c = lax.axis_index("c")
    s = lax.axis_index("s")
    w = c * NS + s

    zv = jnp.zeros((16,), jnp.float32)

    def zb(i, carry):
        zbuf[i // (D // 16), pl.ds((i % (D // 16)) * 16, 16)] = zv
        return carry

    lax.fori_loop(0, ZBUF * (D // 16), zb, 0)

    def zs(i, carry):
        pltpu.sync_copy(zbuf, acc.at[pl.ds(s * ZROWS + i * ZBUF, ZBUF)])
        return carry

    lax.fori_loop(0, ZROWS // ZBUF, zs, 0)
    plsc.subcore_barrier()

    def gather(t, buf, sem):
        pltpu.async_copy(x_hbm.at[sidx.at[t]], buf, sem)

    def gwait(t, buf, sem):
        pltpu.make_async_copy(x_hbm.at[sidx.at[t]], buf, sem).wait()

    def scat(t, buf):
        pltpu.sync_copy(buf, acc.at[didx.at[t]], add=True)

    def section(sec, carry):
        pltpu.sync_copy(src_hbm.at[w, sec], sidx)
        pltpu.sync_copy(dst_hbm.at[w, sec], didx)
        gather(0, rows0, sem0)

        def pair(i, c2):
            t = 2 * i
            gather(t + 1, rows1, sem1)
            gwait(t, rows0, sem0)
            scat(t, rows0)
            gather(t + 2, rows0, sem0)
            gwait(t + 1, rows1, sem1)
            scat(t + 1, rows1)
            return c2

        lax.fori_loop(0, (SECCHUNK - 1) // 2, pair, 0)
        gwait(SECCHUNK - 1, rows0, sem0)
        scat(SECCHUNK - 1, rows0)
        return carry

    lax.fori_loop(0, NSEC, section, 0)
    plsc.subcore_barrier()

    pltpu.sync_copy(acc.at[pl.ds(s * ZROWS, ZROWS)],
                    out_hbm.at[c, pl.ds(s * ZROWS, ZROWS)])


def _sc_agg(x_pad, src3, dst3):
    mesh = plsc.VectorSubcoreMesh(core_axis_name="c", subcore_axis_name="s")
    f = pl.kernel(
        _sc_agg_body,
        out_type=jax.ShapeDtypeStruct((NC, NPAD, D), jnp.float32),
        mesh=mesh,
        scratch_types=[
            pltpu.VMEM((SECCHUNK, CHUNK), jnp.int32),
            pltpu.VMEM((SECCHUNK, CHUNK), jnp.int32),
            pltpu.VMEM((CHUNK, D), jnp.float32),
            pltpu.VMEM((CHUNK, D), jnp.float32),
            pltpu.VMEM((ZBUF, D), jnp.float32),
            pltpu.VMEM_SHARED((NPAD, D), jnp.float32),
            pltpu.SemaphoreType.DMA,
            pltpu.SemaphoreType.DMA,
            pltpu.SemaphoreType.DMA,
        ],
    )
    return f(x_pad, src3, dst3)


def _sortable(s):
    b = jax.lax.bitcast_convert_type(s, jnp.int32)
    return jnp.where(b >= 0, b, b ^ jnp.int32(0x7FFFFFFF))


def _count(m):
    return jnp.sum(m.astype(jnp.int32))


def _radix_kth(key, live, need):
    imin = jnp.int32(INT_MIN)

    def body(i, prefix):
        cand = prefix | (jnp.int32(1) << (jnp.int32(31) - i))
        scand = cand ^ imin
        cnt = _count(live & (key >= scand))
        return jnp.where(cnt >= need, cand, prefix)

    prefix = lax.fori_loop(0, 32, body, jnp.int32(0))
    return prefix ^ imin


def _tc_conv_body(p0, p1, X, Wrel, brel, Wroot, pw, H_o, score_o):
    agg = p0[...] + p1[...]
    H = agg @ Wrel[...] + brel[...] + X[...] @ Wroot[...]
    H = jnp.maximum(H, 0.0)
    H_o[...] = H
    pwv = pw[...]
    rnorm = 1.0 / (jnp.sqrt(jnp.sum(pwv * pwv)) + 1e-16)
    score_o[...] = jnp.tanh((H @ pwv.reshape(D, 1)) * rnorm)


def _tc_conv(p0, p1, X, Wrel, brel, Wroot, pw):
    return pl.pallas_call(
        _tc_conv_body,
        out_shape=[jax.ShapeDtypeStruct((NPAD, D), jnp.float32),
                   jax.ShapeDtypeStruct((NPAD, 1), jnp.float32)],
    )(p0, p1, X, Wrel, brel, Wroot, pw)


def _tc_select_body(nhist, k, *refs):
    (s80, mask80, idx80) = refs[:3]
    hist = refs[3:3 + nhist]
    (s80m_o, sel80_o) = refs[3 + nhist:]

    live = mask80[...] != 0
    s80m = jnp.where(live, s80[...], -jnp.inf)
    s80m_o[...] = s80m

    need = jnp.int32(k)
    sel = jnp.zeros((NROW, D), jnp.bool_)
    for lvl in range(nhist + 1):
        sl = s80m if lvl == 0 else hist[lvl - 1][...]
        key = _sortable(jnp.where(live, sl, -jnp.inf))
        K = _radix_kth(key, live, need)
        gt = live & (key > K)
        sel = sel | gt
        need = need - _count(gt)
        live = live & (key == K)

    idx = idx80[...]

    def ibody(i, lohi):
        lo, hi = lohi
        mid = (lo + hi) // 2
        cnt = _count(live & (idx < mid))
        return (jnp.where(cnt >= need, lo, mid + 1),
                jnp.where(cnt >= need, mid, hi))

    lo, hi = lax.fori_loop(0, 15, ibody, (jnp.int32(0), jnp.int32(NPAD)))
    sel = sel | (live & (idx < lo))
    sel80_o[...] = sel.astype(jnp.float32)


def _tc_select(nhist, k, s80, mask80, idx80, hist):
    body = functools.partial(_tc_select_body, nhist, k)
    return pl.pallas_call(
        body,
        out_shape=[jax.ShapeDtypeStruct((NROW, D), jnp.float32),
                   jax.ShapeDtypeStruct((NROW, D), jnp.float32)],
    )(s80, mask80, idx80, *hist)


def _tc_apply_body(H, score, selc, Xn_o, xmax_o, xsum_o):
    keep = selc[...] != 0.0
    Xn = jnp.where(keep, H[...] * score[...], 0.0)
    Xn_o[...] = Xn
    xmax_o[...] = jnp.max(jnp.where(keep, Xn, -jnp.inf), axis=0, keepdims=True)
    xsum_o[...] = jnp.sum(Xn, axis=0, keepdims=True)


def _tc_apply(H, score, selc):
    return pl.pallas_call(
        _tc_apply_body,
        out_shape=[jax.ShapeDtypeStruct((NPAD, D), jnp.float32),
                   jax.ShapeDtypeStruct((1, D), jnp.float32),
                   jax.ShapeDtypeStruct((1, D), jnp.float32)],
    )(H, score, selc)


def _tc_head_body(hA, hB, L1a, L1b_, L1bias, L2W, L2b, L3W, L3b, out):
    h = hA[...] @ L1a[...] + hB[...] @ L1b_[...] + L1bias[...]
    h = jnp.maximum(h, 0.0)
    h = jnp.maximum(h @ L2W[...] + L2b[...], 0.0)
    out[...] = h @ L3W[...] + L3b[...]


def _tc_head(hA, hB, L1a, L1b_, L1bias, L2W, L2b, L3W, L3b):
    return pl.pallas_call(
        _tc_head_body,
        out_shape=jax.ShapeDtypeStruct((1, 1), jnp.float32),
    )(hA, hB, L1a, L1b_, L1bias, L2W, L2b, L3W, L3b)


def kernel(x, edge_index, batch, W1_rel, b1, W1_root, p1_w, W2_rel, b2,
           W2_root, p2_w, W3_rel, b3, W3_root, p3_w, L1_W, L1_b, L2_W, L2_b,
           L3_W, L3_b):
    src = edge_index[0].reshape(NC * NS, NSEC, SECCHUNK, CHUNK)
    dst = edge_index[1].reshape(NC * NS, NSEC, SECCHUNK, CHUNK)

    X = jnp.pad(x, ((0, NPAD - NNODE), (0, 0)))
    idx_flat = jnp.arange(NPAD, dtype=jnp.int32)
    idx80 = idx_flat.reshape(NROW, D)
    mask80 = (idx_flat < NNODE).astype(jnp.float32).reshape(NROW, D)

    ks = (8000, 6400, 5120)
    rounds = ((W1_rel, b1.reshape(1, D), W1_root, p1_w.reshape(1, D)),
              (W2_rel, b2.reshape(1, D), W2_root, p2_w.reshape(1, D)),
              (W3_rel, b3.reshape(1, D), W3_root, p3_w.reshape(1, D)))

    hist = []
    pooled = []
    for r in range(3):
        Wrel, brel, Wroot, pw = rounds[r]
        parts = _sc_agg(X, src, dst)
        H, score = _tc_conv(parts[0], parts[1], X, Wrel, brel, Wroot, pw)
        s80 = score.reshape(NROW, D)
        s80m, sel80 = _tc_select(r, ks[r], s80, mask80, idx80, hist)
        selc = sel80.reshape(NPAD, 1)
        X, xmax, xsum = _tc_apply(H, score, selc)
        hist = [s80m] + hist
        mask80 = sel80
        pooled.append((xmax, xsum / ks[r]))

    hA = pooled[0][0] + pooled[1][0] + pooled[2][0]
    hB = pooled[0][1] + pooled[1][1] + pooled[2][1]
    out = _tc_head(hA, hB, L1_W[:D], L1_W[D:], L1_b.reshape(1, D),
                   L2_W, L2_b.reshape(1, D // 2), L3_W, L3_b.reshape(1, 1))
    return out.reshape(-1)


# speedup vs baseline: 39.7294x; 1.0144x over previous
import functools
import jax
import jax.numpy as jnp
from jax import lax
from jax.experimental import pallas as pl
from jax.experimental.pallas import tpu as pltpu
from jax.experimental.pallas import tpu_sc as plsc

NNODE = 10000
NPAD = 10240
NROW = NPAD // 128
EDGES = 320000
D = 128
NC, NS = 2, 16
EPW = EDGES // (NC * NS)
CHUNK = 80
NCHUNK = EPW // CHUNK
NSEC = 5
SECCHUNK = NCHUNK // NSEC
ZROWS = NPAD // NS
ZBUF = 32
INT_MIN = -2147483648


def _sc_agg_body(x_hbm, src_hbm, dst_hbm, out_hbm,
                 sidx, didx, rows0, rows1, zbuf, acc, isem, sem0, sem1):
    c = lax.axis_index("c")
    s = lax.axis_index("s")
    w = c * NS + s

    zv = jnp.zeros((16,), jnp.float32)

    def zb(i, carry):
        zbuf[i // (D // 16), pl.ds((i % (D // 16)) * 16, 16)] = zv
        return carry

    lax.fori_loop(0, ZBUF * (D // 16), zb, 0)

    def zs(i, carry):
        pltpu.sync_copy(zbuf, acc.at[pl.ds(s * ZROWS + i * ZBUF, ZBUF)])
        return carry

    lax.fori_loop(0, ZROWS // ZBUF, zs, 0)
    plsc.subcore_barrier()

    def gather(t, buf, sem):
        pltpu.async_copy(x_hbm.at[sidx.at[t]], buf, sem)

    def gwait(t, buf, sem):
        pltpu.make_async_copy(x_hbm.at[sidx.at[t]], buf, sem).wait()

    def scat(t, buf):
        pltpu.sync_copy(buf, acc.at[didx.at[t]], add=True)

    def section(sec, carry):
        pltpu.sync_copy(src_hbm.at[w, sec], sidx)
        pltpu.sync_copy(dst_hbm.at[w, sec], didx)
        gather(0, rows0, sem0)

        def pair(i, c2):
            t = 2 * i
            gather(t + 1, rows1, sem1)
            gwait(t, rows0, sem0)
            scat(t, rows0)
            gather(t + 2, rows0, sem0)
            gwait(t + 1, rows1, sem1)
            scat(t + 1, rows1)
            return c2

        lax.fori_loop(0, (SECCHUNK - 1) // 2, pair, 0)
        gwait(SECCHUNK - 1, rows0, sem0)
        scat(SECCHUNK - 1, rows0)
        return carry

    lax.fori_loop(0, NSEC, section, 0)
    plsc.subcore_barrier()

    pltpu.sync_copy(acc.at[pl.ds(s * ZROWS, ZROWS)],
                    out_hbm.at[c, pl.ds(s * ZROWS, ZROWS)])


def _sc_agg(x_pad, src3, dst3):
    mesh = plsc.VectorSubcoreMesh(core_axis_name="c", subcore_axis_name="s")
    f = pl.kernel(
        _sc_agg_body,
        out_type=jax.ShapeDtypeStruct((NC, NPAD, D), jnp.float32),
        mesh=mesh,
        scratch_types=[
            pltpu.VMEM((SECCHUNK, CHUNK), jnp.int32),
            pltpu.VMEM((SECCHUNK, CHUNK), jnp.int32),
            pltpu.VMEM((CHUNK, D), jnp.float32),
            pltpu.VMEM((CHUNK, D), jnp.float32),
            pltpu.VMEM((ZBUF, D), jnp.float32),
            pltpu.VMEM_SHARED((NPAD, D), jnp.float32),
            pltpu.SemaphoreType.DMA,
            pltpu.SemaphoreType.DMA,
            pltpu.SemaphoreType.DMA,
        ],
    )
    return f(x_pad, src3, dst3)


def _sortable(s):
    b = jax.lax.bitcast_convert_type(s, jnp.int32)
    return jnp.where(b >= 0, b, b ^ jnp.int32(0x7FFFFFFF))


def _count(m):
    return jnp.sum(m.astype(jnp.int32))


def _radix_kth(key, live, need):
    imin = jnp.int32(INT_MIN)

    def body(i, prefix):
        cand = prefix | (jnp.int32(1) << (jnp.int32(31) - i))
        scand = cand ^ imin
        cnt = _count(live & (key >= scand))
        return jnp.where(cnt >= need, cand, prefix)

    prefix = lax.fori_loop(0, 32, body, jnp.int32(0))
    return prefix ^ imin


def _tc_conv_body(p0, p1, X, Wrel, brel, Wroot, pw, H_o, score_o):
    agg = p0[pl.ds(0, NNODE), :] + p1[pl.ds(0, NNODE), :]
    H = agg @ Wrel[...] + brel[...] + X[...] @ Wroot[...]
    H = jnp.maximum(H, 0.0)
    H_o[...] = H
    pwv = pw[...]
    rnorm = 1.0 / (jnp.sqrt(jnp.sum(pwv * pwv)) + 1e-16)
    score_o[...] = jnp.tanh((H @ pwv.reshape(D, 1)) * rnorm)


def _tc_conv(p0, p1, X, Wrel, brel, Wroot, pw):
    return pl.pallas_call(
        _tc_conv_body,
        out_shape=[jax.ShapeDtypeStruct((NNODE, D), jnp.float32),
                   jax.ShapeDtypeStruct((NNODE, 1), jnp.float32)],
    )(p0, p1, X, Wrel, brel, Wroot, pw)


def _idx80():
    return (lax.broadcasted_iota(jnp.int32, (NROW, D), 0) * D +
            lax.broadcasted_iota(jnp.int32, (NROW, D), 1))


def _tc_select_body(nhist, k, *refs):
    s80 = refs[0]
    if nhist == 0:
        hist = ()
        (s80m_o, sel80_o) = refs[1:]
        live = _idx80() < NNODE
    else:
        mask80 = refs[1]
        hist = refs[2:2 + nhist]
        (s80m_o, sel80_o) = refs[2 + nhist:]
        live = mask80[...] != 0
    s80m = jnp.where(live, s80[...], -jnp.inf)
    s80m_o[...] = s80m

    need = jnp.int32(k)
    sel = jnp.zeros((NROW, D), jnp.bool_)
    for lvl in range(nhist + 1):
        sl = s80m if lvl == 0 else hist[lvl - 1][...]
        key = _sortable(jnp.where(live, sl, -jnp.inf))
        K = _radix_kth(key, live, need)
        gt = live & (key > K)
        sel = sel | gt
        need = need - _count(gt)
        live = live & (key == K)

    idx = _idx80()

    def ibody(i, lohi):
        lo, hi = lohi
        mid = (lo + hi) // 2
        cnt = _count(live & (idx < mid))
        return (jnp.where(cnt >= need, lo, mid + 1),
                jnp.where(cnt >= need, mid, hi))

    lo, hi = lax.fori_loop(0, 15, ibody, (jnp.int32(0), jnp.int32(NPAD)))
    sel = sel | (live & (idx < lo))
    sel80_o[...] = sel.astype(jnp.float32)


def _tc_select(nhist, k, s80, mask80, hist):
    body = functools.partial(_tc_select_body, nhist, k)
    args = (s80,) if nhist == 0 else (s80, mask80, *hist)
    return pl.pallas_call(
        body,
        out_shape=[jax.ShapeDtypeStruct((NROW, D), jnp.float32),
                   jax.ShapeDtypeStruct((NROW, D), jnp.float32)],
    )(*args)


def _tc_apply_body(H, score, selc, Xn_o, xmax_o, xsum_o):
    keep = selc[...] != 0.0
    Xn = jnp.where(keep, H[...] * score[...], 0.0)
    Xn_o[...] = Xn
    xmax_o[...] = jnp.max(jnp.where(keep, Xn, -jnp.inf), axis=0, keepdims=True)
    xsum_o[...] = jnp.sum(Xn, axis=0, keepdims=True)


def _tc_apply(H, score, selc):
    return pl.pallas_call(
        _tc_apply_body,
        out_shape=[jax.ShapeDtypeStruct((NNODE, D), jnp.float32),
                   jax.ShapeDtypeStruct((1, D), jnp.float32),
                   jax.ShapeDtypeStruct((1, D), jnp.float32)],
    )(H, score, selc)


def _tc_head_body(m1, s1, m2, s2, m3, s3,
                  L1a, L1b_, L1bias, L2W, L2b, L3W, L3b, out):
    hA = m1[...] + m2[...] + m3[...]
    hB = s1[...] / 8000.0 + s2[...] / 6400.0 + s3[...] / 5120.0
    h = hA @ L1a[...] + hB @ L1b_[...] + L1bias[...]
    h = jnp.maximum(h, 0.0)
    h = jnp.maximum(h @ L2W[...] + L2b[...], 0.0)
    out[...] = h @ L3W[...] + L3b[...]


def _tc_head(pooled, L1a, L1b_, L1bias, L2W, L2b, L3W, L3b):
    (m1, s1), (m2, s2), (m3, s3) = pooled
    return pl.pallas_call(
        _tc_head_body,
        out_shape=jax.ShapeDtypeStruct((1, 1), jnp.float32),
    )(m1, s1, m2, s2, m3, s3, L1a, L1b_, L1bias, L2W, L2b, L3W, L3b)


def kernel(x, edge_index, batch, W1_rel, b1, W1_root, p1_w, W2_rel, b2,
           W2_root, p2_w, W3_rel, b3, W3_root, p3_w, L1_W, L1_b, L2_W, L2_b,
           L3_W, L3_b):
    src = edge_index[0].reshape(NC * NS, NSEC, SECCHUNK, CHUNK)
    dst = edge_index[1].reshape(NC * NS, NSEC, SECCHUNK, CHUNK)

    X = x
    ks = (8000, 6400, 5120)
    rounds = ((W1_rel, b1.reshape(1, D), W1_root, p1_w.reshape(1, D)),
              (W2_rel, b2.reshape(1, D), W2_root, p2_w.reshape(1, D)),
              (W3_rel, b3.reshape(1, D), W3_root, p3_w.reshape(1, D)))

    hist = []
    pooled = []
    mask80 = None
    for r in range(3):
        Wrel, brel, Wroot, pw = rounds[r]
        parts = _sc_agg(X, src, dst)
        H, score = _tc_conv(parts[0], parts[1], X, Wrel, brel, Wroot, pw)
        s80 = jnp.pad(score, ((0, NPAD - NNODE), (0, 0))).reshape(NROW, D)
        s80m, sel80 = _tc_select(r, ks[r], s80, mask80, hist)
        selc = sel80.reshape(NPAD, 1)[:NNODE]
        X, xmax, xsum = _tc_apply(H, score, selc)
        hist = [s80m] + hist
        mask80 = sel80
        pooled.append((xmax, xsum))

    out = _tc_head(pooled, L1_W[:D], L1_W[D:], L1_b.reshape(1, D),
                   L2_W, L2_b.reshape(1, D // 2), L3_W, L3_b.reshape(1, 1))
    return out.reshape(-1)


# speedup vs baseline: 39.9268x; 1.0050x over previous
import functools
import jax
import jax.numpy as jnp
from jax import lax
from jax.experimental import pallas as pl
from jax.experimental.pallas import tpu as pltpu
from jax.experimental.pallas import tpu_sc as plsc

NNODE = 10000
NPAD = 10240
NROW = NPAD // 128
EDGES = 320000
D = 128
NC, NS = 2, 16
EPW = EDGES // (NC * NS)
CHUNK = 80
NCHUNK = EPW // CHUNK
NSEC = 5
SECCHUNK = NCHUNK // NSEC
ZROWS = NPAD // NS
ZBUF = 32
INT_MIN = -2147483648


def _sc_agg_body(x_hbm, src_hbm, dst_hbm, out_hbm,
                 sidx, didx, rows0, rows1, zbuf, acc, isem, sem0, sem1):
    c = lax.axis_index("c")
    s = lax.axis_index("s")
    w = c * NS + s

    zv = jnp.zeros((16,), jnp.float32)

    def zb(i, carry):
        zbuf[i // (D // 16), pl.ds((i % (D // 16)) * 16, 16)] = zv
        return carry

    lax.fori_loop(0, ZBUF * (D // 16), zb, 0)

    def zs(i, carry):
        pltpu.sync_copy(zbuf, acc.at[pl.ds(s * ZROWS + i * ZBUF, ZBUF)])
        return carry

    lax.fori_loop(0, ZROWS // ZBUF, zs, 0)
    plsc.subcore_barrier()

    def gather(t, buf, sem):
        pltpu.async_copy(x_hbm.at[sidx.at[t]], buf, sem)

    def gwait(t, buf, sem):
        pltpu.make_async_copy(x_hbm.at[sidx.at[t]], buf, sem).wait()

    def scat(t, buf):
        pltpu.sync_copy(buf, acc.at[didx.at[t]], add=True)

    def section(sec, carry):
        pltpu.sync_copy(src_hbm.at[w, sec], sidx)
        pltpu.sync_copy(dst_hbm.at[w, sec], didx)
        gather(0, rows0, sem0)

        def pair(i, c2):
            t = 2 * i
            gather(t + 1, rows1, sem1)
            gwait(t, rows0, sem0)
            scat(t, rows0)
            gather(t + 2, rows0, sem0)
            gwait(t + 1, rows1, sem1)
            scat(t + 1, rows1)
            return c2

        lax.fori_loop(0, (SECCHUNK - 1) // 2, pair, 0)
        gwait(SECCHUNK - 1, rows0, sem0)
        scat(SECCHUNK - 1, rows0)
        return carry

    lax.fori_loop(0, NSEC, section, 0)
    plsc.subcore_barrier()

    pltpu.sync_copy(acc.at[pl.ds(s * ZROWS, ZROWS)],
                    out_hbm.at[c, pl.ds(s * ZROWS, ZROWS)])


def _sc_agg(x_pad, src3, dst3):
    mesh = plsc.VectorSubcoreMesh(core_axis_name="c", subcore_axis_name="s")
    f = pl.kernel(
        _sc_agg_body,
        out_type=jax.ShapeDtypeStruct((NC, NPAD, D), jnp.float32),
        mesh=mesh,
        scratch_types=[
            pltpu.VMEM((SECCHUNK, CHUNK), jnp.int32),
            pltpu.VMEM((SECCHUNK, CHUNK), jnp.int32),
            pltpu.VMEM((CHUNK, D), jnp.float32),
            pltpu.VMEM((CHUNK, D), jnp.float32),
            pltpu.VMEM((ZBUF, D), jnp.float32),
            pltpu.VMEM_SHARED((NPAD, D), jnp.float32),
            pltpu.SemaphoreType.DMA,
            pltpu.SemaphoreType.DMA,
            pltpu.SemaphoreType.DMA,
        ],
    )
    return f(x_pad, src3, dst3)


def _sortable(s):
    b = jax.lax.bitcast_convert_type(s, jnp.int32)
    return jnp.where(b >= 0, b, b ^ jnp.int32(0x7FFFFFFF))


def _count(m):
    return jnp.sum(m.astype(jnp.int32))


def _radix_kth(key, live, need):
    imin = jnp.int32(INT_MIN)

    def body(i, prefix):
        cand = prefix | (jnp.int32(1) << (jnp.int32(31) - i))
        scand = cand ^ imin
        cnt = _count(live & (key >= scand))
        return jnp.where(cnt >= need, cand, prefix)

    prefix = lax.fori_loop(0, 32, body, jnp.int32(0))
    return prefix ^ imin


def _tc_conv_body(p0, p1, X, Wrel, brel, Wroot, pw, H_o, score_o):
    agg = p0[pl.ds(0, NNODE), :] + p1[pl.ds(0, NNODE), :]
    H = agg @ Wrel[...] + brel[...] + X[...] @ Wroot[...]
    H = jnp.maximum(H, 0.0)
    H_o[...] = H
    pwv = pw[...]
    rnorm = 1.0 / (jnp.sqrt(jnp.sum(pwv * pwv)) + 1e-16)
    score_o[...] = jnp.tanh((H @ pwv.reshape(D, 1)) * rnorm)


def _tc_conv(p0, p1, X, Wrel, brel, Wroot, pw):
    return pl.pallas_call(
        _tc_conv_body,
        out_shape=[jax.ShapeDtypeStruct((NNODE, D), jnp.float32),
                   jax.ShapeDtypeStruct((NNODE, 1), jnp.float32)],
    )(p0, p1, X, Wrel, brel, Wroot, pw)


def _idx80():
    return (lax.broadcasted_iota(jnp.int32, (NROW, D), 0) * D +
            lax.broadcasted_iota(jnp.int32, (NROW, D), 1))


def _tc_select_body(nhist, k, *refs):
    s80 = refs[0]
    if nhist == 0:
        hist = ()
        (s80m_o, sel80_o) = refs[1:]
        live = _idx80() < NNODE
    else:
        mask80 = refs[1]
        hist = refs[2:2 + nhist]
        (s80m_o, sel80_o) = refs[2 + nhist:]
        live = mask80[...] != 0
    s80m = jnp.where(live, s80[...], -jnp.inf)
    s80m_o[...] = s80m

    need = jnp.int32(k)
    sel = jnp.zeros((NROW, D), jnp.bool_)
    for lvl in range(nhist + 1):
        sl = s80m if lvl == 0 else hist[lvl - 1][...]
        key = _sortable(jnp.where(live, sl, -jnp.inf))
        K = _radix_kth(key, live, need)
        gt = live & (key > K)
        sel = sel | gt
        need = need - _count(gt)
        live = live & (key == K)

    idx = _idx80()

    def ibody(i, lohi):
        lo, hi = lohi
        mid = (lo + hi) // 2
        cnt = _count(live & (idx < mid))
        return (jnp.where(cnt >= need, lo, mid + 1),
                jnp.where(cnt >= need, mid, hi))

    lo, hi = lax.fori_loop(0, 15, ibody, (jnp.int32(0), jnp.int32(NPAD)))
    sel = sel | (live & (idx < lo))
    sel80_o[...] = sel.astype(jnp.float32)


def _tc_select(nhist, k, s80, mask80, hist):
    body = functools.partial(_tc_select_body, nhist, k)
    args = (s80,) if nhist == 0 else (s80, mask80, *hist)
    return pl.pallas_call(
        body,
        out_shape=[jax.ShapeDtypeStruct((NROW, D), jnp.float32),
                   jax.ShapeDtypeStruct((NROW, D), jnp.float32)],
    )(*args)


def _tc_apply_body(H, score, selc, Xn_o, xmax_o, xsum_o):
    keep = selc[...] != 0.0
    Xn = jnp.where(keep, H[...] * score[...], 0.0)
    Xn_o[...] = Xn
    xmax_o[...] = jnp.max(jnp.where(keep, Xn, -jnp.inf), axis=0, keepdims=True)
    xsum_o[...] = jnp.sum(Xn, axis=0, keepdims=True)


def _tc_apply(H, score, selc):
    return pl.pallas_call(
        _tc_apply_body,
        out_shape=[jax.ShapeDtypeStruct((NNODE, D), jnp.float32),
                   jax.ShapeDtypeStruct((1, D), jnp.float32),
                   jax.ShapeDtypeStruct((1, D), jnp.float32)],
    )(H, score, selc)


def _tc_final_body(H, score, selc, m1, s1, m2, s2,
                   L1a, L1b_, L1bias, L2W, L2b, L3W, L3b, out):
    keep = selc[...] != 0.0
    Xn = jnp.where(keep, H[...] * score[...], 0.0)
    m3 = jnp.max(jnp.where(keep, Xn, -jnp.inf), axis=0, keepdims=True)
    s3 = jnp.sum(Xn, axis=0, keepdims=True)
    hA = m1[...] + m2[...] + m3
    hB = s1[...] / 8000.0 + s2[...] / 6400.0 + s3 / 5120.0
    h = hA @ L1a[...] + hB @ L1b_[...] + L1bias[...]
    h = jnp.maximum(h, 0.0)
    h = jnp.maximum(h @ L2W[...] + L2b[...], 0.0)
    out[...] = h @ L3W[...] + L3b[...]


def _tc_final(H, score, selc, pooled, L1a, L1b_, L1bias, L2W, L2b, L3W, L3b):
    (m1, s1), (m2, s2) = pooled
    return pl.pallas_call(
        _tc_final_body,
        out_shape=jax.ShapeDtypeStruct((1, 1), jnp.float32),
    )(H, score, selc, m1, s1, m2, s2,
      L1a, L1b_, L1bias, L2W, L2b, L3W, L3b)


def kernel(x, edge_index, batch, W1_rel, b1, W1_root, p1_w, W2_rel, b2,
           W2_root, p2_w, W3_rel, b3, W3_root, p3_w, L1_W, L1_b, L2_W, L2_b,
           L3_W, L3_b):
    src = edge_index[0].reshape(NC * NS, NSEC, SECCHUNK, CHUNK)
    dst = edge_index[1].reshape(NC * NS, NSEC, SECCHUNK, CHUNK)

    X = x
    ks = (8000, 6400, 5120)
    rounds = ((W1_rel, b1.reshape(1, D), W1_root, p1_w.reshape(1, D)),
              (W2_rel, b2.reshape(1, D), W2_root, p2_w.reshape(1, D)),
              (W3_rel, b3.reshape(1, D), W3_root, p3_w.reshape(1, D)))

    hist = []
    pooled = []
    mask80 = None
    for r in range(3):
        Wrel, brel, Wroot, pw = rounds[r]
        parts = _sc_agg(X, src, dst)
        H, score = _tc_conv(parts[0], parts[1], X, Wrel, brel, Wroot, pw)
        s80 = jnp.pad(score, ((0, NPAD - NNODE), (0, 0))).reshape(NROW, D)
        s80m, sel80 = _tc_select(r, ks[r], s80, mask80, hist)
        selc = sel80.reshape(NPAD, 1)[:NNODE]
        if r < 2:
            X, xmax, xsum = _tc_apply(H, score, selc)
            pooled.append((xmax, xsum))
        hist = [s80m] + hist
        mask80 = sel80

    out = _tc_final(H, score, selc, pooled,
                    L1_W[:D], L1_W[D:], L1_b.reshape(1, D),
                    L2_W, L2_b.reshape(1, D // 2), L3_W, L3_b.reshape(1, 1))
    return out.reshape(-1)


# speedup vs baseline: 41.3821x; 1.0364x over previous
import functools
import jax
import jax.numpy as jnp
from jax import lax
from jax.experimental import pallas as pl
from jax.experimental.pallas import tpu as pltpu
from jax.experimental.pallas import tpu_sc as plsc

NNODE = 10000
NPAD = 10240
NROW = NPAD // 128
EDGES = 320000
D = 128
NC, NS = 2, 16
EPW = EDGES // (NC * NS)
CHUNK = 80
NCHUNK = EPW // CHUNK
NSEC = 5
SECCHUNK = NCHUNK // NSEC
ZROWS = NPAD // NS
ZBUF = 32
INT_MIN = -2147483648


def _sc_agg_body(x_hbm, src_hbm, dst_hbm, out_hbm,
                 sidxA, didxA, sidxB, didxB, rows0, rows1, zbuf, acc,
                 isem, zsem, sem0, sem1):
    c = lax.axis_index("c")
    s = lax.axis_index("s")
    w = c * NS + s
    slabs = ((sidxA, didxA), (sidxB, didxB))

    def slab_load(sec, bufs):
        pltpu.async_copy(src_hbm.at[w, sec], bufs[0], isem)
        pltpu.async_copy(dst_hbm.at[w, sec], bufs[1], isem)

    def slab_wait(sec, bufs):
        pltpu.make_async_copy(src_hbm.at[w, sec], bufs[0], isem).wait()
        pltpu.make_async_copy(dst_hbm.at[w, sec], bufs[1], isem).wait()

    slab_load(0, slabs[0])

    zv = jnp.zeros((16,), jnp.float32)

    def zb(i, carry):
        zbuf[i // (D // 16), pl.ds((i % (D // 16)) * 16, 16)] = zv
        return carry

    lax.fori_loop(0, ZBUF * (D // 16), zb, 0)

    nz = ZROWS // ZBUF
    for half in range(2):
        for i in range(half * nz // 2, (half + 1) * nz // 2):
            pltpu.async_copy(zbuf, acc.at[pl.ds(s * ZROWS + i * ZBUF, ZBUF)],
                             zsem)
        for i in range(half * nz // 2, (half + 1) * nz // 2):
            pltpu.make_async_copy(
                zbuf, acc.at[pl.ds(s * ZROWS + i * ZBUF, ZBUF)], zsem).wait()
    slab_wait(0, slabs[0])
    plsc.subcore_barrier()

    for sec in range(NSEC):
        sidx, didx = slabs[sec % 2]
        if sec + 1 < NSEC:
            slab_load(sec + 1, slabs[(sec + 1) % 2])

        def gather(t, buf, sem):
            pltpu.async_copy(x_hbm.at[sidx.at[t]], buf, sem)

        def gwait(t, buf, sem):
            pltpu.make_async_copy(x_hbm.at[sidx.at[t]], buf, sem).wait()

        def scat(t, buf):
            pltpu.sync_copy(buf, acc.at[didx.at[t]], add=True)

        gather(0, rows0, sem0)

        def pair(i, c2):
            t = 2 * i
            gather(t + 1, rows1, sem1)
            gwait(t, rows0, sem0)
            scat(t, rows0)
            gather(t + 2, rows0, sem0)
            gwait(t + 1, rows1, sem1)
            scat(t + 1, rows1)
            return c2

        lax.fori_loop(0, (SECCHUNK - 1) // 2, pair, 0)
        gwait(SECCHUNK - 1, rows0, sem0)
        scat(SECCHUNK - 1, rows0)
        if sec + 1 < NSEC:
            slab_wait(sec + 1, slabs[(sec + 1) % 2])

    plsc.subcore_barrier()

    pltpu.sync_copy(acc.at[pl.ds(s * ZROWS, ZROWS)],
                    out_hbm.at[c, pl.ds(s * ZROWS, ZROWS)])


def _sc_agg(x_pad, src3, dst3):
    mesh = plsc.VectorSubcoreMesh(core_axis_name="c", subcore_axis_name="s")
    f = pl.kernel(
        _sc_agg_body,
        out_type=jax.ShapeDtypeStruct((NC, NPAD, D), jnp.float32),
        mesh=mesh,
        scratch_types=[
            pltpu.VMEM((SECCHUNK, CHUNK), jnp.int32),
            pltpu.VMEM((SECCHUNK, CHUNK), jnp.int32),
            pltpu.VMEM((SECCHUNK, CHUNK), jnp.int32),
            pltpu.VMEM((SECCHUNK, CHUNK), jnp.int32),
            pltpu.VMEM((CHUNK, D), jnp.float32),
            pltpu.VMEM((CHUNK, D), jnp.float32),
            pltpu.VMEM((ZBUF, D), jnp.float32),
            pltpu.VMEM_SHARED((NPAD, D), jnp.float32),
            pltpu.SemaphoreType.DMA,
            pltpu.SemaphoreType.DMA,
            pltpu.SemaphoreType.DMA,
            pltpu.SemaphoreType.DMA,
        ],
    )
    return f(x_pad, src3, dst3)


def _sortable(s):
    b = jax.lax.bitcast_convert_type(s, jnp.int32)
    return jnp.where(b >= 0, b, b ^ jnp.int32(0x7FFFFFFF))


def _count(m):
    return jnp.sum(m.astype(jnp.int32))


def _radix_kth(key, live, need):
    imin = jnp.int32(INT_MIN)

    def body(i, prefix):
        cand = prefix | (jnp.int32(1) << (jnp.int32(31) - i))
        scand = cand ^ imin
        cnt = _count(live & (key >= scand))
        return jnp.where(cnt >= need, cand, prefix)

    prefix = lax.fori_loop(0, 32, body, jnp.int32(0))
    return prefix ^ imin


def _tc_conv_body(p0, p1, X, Wrel, brel, Wroot, pw, H_o, score_o):
    agg = p0[pl.ds(0, NNODE), :] + p1[pl.ds(0, NNODE), :]
    H = agg @ Wrel[...] + brel[...] + X[...] @ Wroot[...]
    H = jnp.maximum(H, 0.0)
    H_o[...] = H
    pwv = pw[...]
    rnorm = 1.0 / (jnp.sqrt(jnp.sum(pwv * pwv)) + 1e-16)
    score_o[...] = jnp.tanh((H @ pwv.reshape(D, 1)) * rnorm)


def _tc_conv(p0, p1, X, Wrel, brel, Wroot, pw):
    return pl.pallas_call(
        _tc_conv_body,
        out_shape=[jax.ShapeDtypeStruct((NNODE, D), jnp.float32),
                   jax.ShapeDtypeStruct((NNODE, 1), jnp.float32)],
    )(p0, p1, X, Wrel, brel, Wroot, pw)


def _idx80():
    return (lax.broadcasted_iota(jnp.int32, (NROW, D), 0) * D +
            lax.broadcasted_iota(jnp.int32, (NROW, D), 1))


def _tc_select_body(nhist, k, *refs):
    s80 = refs[0]
    if nhist == 0:
        hist = ()
        (s80m_o, sel80_o) = refs[1:]
        live = _idx80() < NNODE
    else:
        mask80 = refs[1]
        hist = refs[2:2 + nhist]
        (s80m_o, sel80_o) = refs[2 + nhist:]
        live = mask80[...] != 0
    s80m = jnp.where(live, s80[...], -jnp.inf)
    s80m_o[...] = s80m

    need = jnp.int32(k)
    sel = jnp.zeros((NROW, D), jnp.bool_)
    for lvl in range(nhist + 1):
        sl = s80m if lvl == 0 else hist[lvl - 1][...]
        key = _sortable(jnp.where(live, sl, -jnp.inf))
        K = _radix_kth(key, live, need)
        gt = live & (key > K)
        sel = sel | gt
        need = need - _count(gt)
        live = live & (key == K)

    idx = _idx80()

    def ibody(i, lohi):
        lo, hi = lohi
        mid = (lo + hi) // 2
        cnt = _count(live & (idx < mid))
        return (jnp.where(cnt >= need, lo, mid + 1),
                jnp.where(cnt >= need, mid, hi))

    lo, hi = lax.fori_loop(0, 15, ibody, (jnp.int32(0), jnp.int32(NPAD)))
    sel = sel | (live & (idx < lo))
    sel80_o[...] = sel.astype(jnp.float32)


def _tc_select(nhist, k, s80, mask80, hist):
    body = functools.partial(_tc_select_body, nhist, k)
    args = (s80,) if nhist == 0 else (s80, mask80, *hist)
    return pl.pallas_call(
        body,
        out_shape=[jax.ShapeDtypeStruct((NROW, D), jnp.float32),
                   jax.ShapeDtypeStruct((NROW, D), jnp.float32)],
    )(*args)


def _tc_apply_body(H, score, selc, Xn_o, xmax_o, xsum_o):
    keep = selc[...] != 0.0
    Xn = jnp.where(keep, H[...] * score[...], 0.0)
    Xn_o[...] = Xn
    xmax_o[...] = jnp.max(jnp.where(keep, Xn, -jnp.inf), axis=0, keepdims=True)
    xsum_o[...] = jnp.sum(Xn, axis=0, keepdims=True)


def _tc_apply(H, score, selc):
    return pl.pallas_call(
        _tc_apply_body,
        out_shape=[jax.ShapeDtypeStruct((NNODE, D), jnp.float32),
                   jax.ShapeDtypeStruct((1, D), jnp.float32),
                   jax.ShapeDtypeStruct((1, D), jnp.float32)],
    )(H, score, selc)


def _tc_final_body(H, score, selc, m1, s1, m2, s2,
                   L1a, L1b_, L1bias, L2W, L2b, L3W, L3b, out):
    keep = selc[...] != 0.0
    Xn = jnp.where(keep, H[...] * score[...], 0.0)
    m3 = jnp.max(jnp.where(keep, Xn, -jnp.inf), axis=0, keepdims=True)
    s3 = jnp.sum(Xn, axis=0, keepdims=True)
    hA = m1[...] + m2[...] + m3
    hB = s1[...] / 8000.0 + s2[...] / 6400.0 + s3 / 5120.0
    h = hA @ L1a[...] + hB @ L1b_[...] + L1bias[...]
    h = jnp.maximum(h, 0.0)
    h = jnp.maximum(h @ L2W[...] + L2b[...], 0.0)
    out[...] = h @ L3W[...] + L3b[...]


def _tc_final(H, score, selc, pooled, L1a, L1b_, L1bias, L2W, L2b, L3W, L3b):
    (m1, s1), (m2, s2) = pooled
    return pl.pallas_call(
        _tc_final_body,
        out_shape=jax.ShapeDtypeStruct((1, 1), jnp.float32),
    )(H, score, selc, m1, s1, m2, s2,
      L1a, L1b_, L1bias, L2W, L2b, L3W, L3b)


def kernel(x, edge_index, batch, W1_rel, b1, W1_root, p1_w, W2_rel, b2,
           W2_root, p2_w, W3_rel, b3, W3_root, p3_w, L1_W, L1_b, L2_W, L2_b,
           L3_W, L3_b):
    src = edge_index[0].reshape(NC * NS, NSEC, SECCHUNK, CHUNK)
    dst = edge_index[1].reshape(NC * NS, NSEC, SECCHUNK, CHUNK)

    X = x
    ks = (8000, 6400, 5120)
    rounds = ((W1_rel, b1.reshape(1, D), W1_root, p1_w.reshape(1, D)),
              (W2_rel, b2.reshape(1, D), W2_root, p2_w.reshape(1, D)),
              (W3_rel, b3.reshape(1, D), W3_root, p3_w.reshape(1, D)))

    hist = []
    pooled = []
    mask80 = None
    for r in range(3):
        Wrel, brel, Wroot, pw = rounds[r]
        parts = _sc_agg(X, src, dst)
        H, score = _tc_conv(parts[0], parts[1], X, Wrel, brel, Wroot, pw)
        s80 = jnp.pad(score, ((0, NPAD - NNODE), (0, 0))).reshape(NROW, D)
        s80m, sel80 = _tc_select(r, ks[r], s80, mask80, hist)
        selc = sel80.reshape(NPAD, 1)[:NNODE]
        if r < 2:
            X, xmax, xsum = _tc_apply(H, score, selc)
            pooled.append((xmax, xsum))
        hist = [s80m] + hist
        mask80 = sel80

    out = _tc_final(H, score, selc, pooled,
                    L1_W[:D], L1_W[D:], L1_b.reshape(1, D),
                    L2_W, L2_b.reshape(1, D // 2), L3_W, L3_b.reshape(1, 1))
    return out.reshape(-1)
